# baseline (speedup 1.0000x reference)
import jax
import jax.numpy as jnp
from jax.experimental import pallas as pl


def _head_body(xa_ref, xb_ref, fc1w_ref, fc1b_ref, fc2w_ref, fc2b_ref,
               outw_ref, outb_ref, o_ref):
    xc = xa_ref[...] + xb_ref[...]
    xc = jnp.maximum(jnp.dot(xc, fc1w_ref[...],
                             preferred_element_type=jnp.float32)
                     + fc1b_ref[...], 0.0)
    xc = jnp.maximum(jnp.dot(xc, fc2w_ref[...],
                             preferred_element_type=jnp.float32)
                     + fc2b_ref[...], 0.0)
    o_ref[...] = (jnp.dot(xc, outw_ref[...],
                          preferred_element_type=jnp.float32)
                  + outb_ref[...])


def _head(xa, xb, fc1_W, fc1_b, fc2_W, fc2_b, out_W, out_b):
    G = xa.shape[0]
    outw_p = jnp.zeros((256, 128), jnp.float32).at[:, :2].set(out_W)
    outb_p = jnp.zeros((128,), jnp.float32).at[:2].set(out_b)
    res = pl.pallas_call(
        _head_body,
        out_shape=jax.ShapeDtypeStruct((G, 128), jnp.float32),
    )(xa, xb, fc1_W, fc1_b.reshape(1, -1), fc2_W, fc2_b.reshape(1, -1),
      outw_p, outb_p.reshape(1, -1))
    return res[:, :2]


def _bn(x, g, b):
    m = jnp.mean(x, axis=0)
    v = jnp.var(x, axis=0)
    return (x - m) / jnp.sqrt(v + 1e-5) * g + b


def _gin_branch(x, edge_index, batch, p, i_branch=None):
    src, dst = edge_index[0], edge_index[1]
    h = x
    for i in range(5):
        agg = jnp.zeros_like(h).at[dst].add(h[src])
        z = h + agg
        z = jnp.maximum(z @ p[f'cW1_{i}'] + p[f'cb1_{i}'], 0.0)
        z = z @ p[f'cW2_{i}'] + p[f'cb2_{i}']
        h = jnp.maximum(z, 0.0)
        h = _bn(h, p[f'bng_{i}'], p[f'bnb_{i}'])
    pooled = jax.ops.segment_sum(h, batch, num_segments=256)
    return jnp.maximum(pooled @ p['fcxd_W'] + p['fcxd_b'], 0.0)


def kernel(x_a, edge_index_a, batch_a, x_b, edge_index_b, batch_b,
           cW1_0, cb1_0, cW2_0, cb2_0, bng_0, bnb_0,
           cW1_1, cb1_1, cW2_1, cb2_1, bng_1, bnb_1,
           cW1_2, cb1_2, cW2_2, cb2_2, bng_2, bnb_2,
           cW1_3, cb1_3, cW2_3, cb2_3, bng_3, bnb_3,
           cW1_4, cb1_4, cW2_4, cb2_4, bng_4, bnb_4,
           fcxd_W, fcxd_b, fc1_W, fc1_b, fc2_W, fc2_b,
           out_W, out_b):
    kw = dict(locals())
    xa = _gin_branch(x_a, edge_index_a, batch_a, kw)
    xb = _gin_branch(x_b, edge_index_b, batch_b, kw)
    return _head(xa, xb, fc1_W, fc1_b, fc2_W, fc2_b, out_W, out_b)


# speedup vs baseline: 8.4492x; 8.4492x over previous
import functools

import jax
import jax.numpy as jnp
from jax import lax
from jax.experimental import pallas as pl
from jax.experimental.pallas import tpu as pltpu
from jax.experimental.pallas import tpu_sc as plsc

N = 100000
E = 1600000
G = 256

_B = 128
_MROWS = 8
_NSUB = 16
_NCORE = 2
_ROWS_PT = 784
_EROWS = _ROWS_PT * _NSUB
_EPAD = _EROWS * _B
_NMACRO = _ROWS_PT // _MROWS
_ZCH = 391
_NZC = 16
_NSH = _ZCH * _NZC * _NSUB


def _agg_body(h2, src2, dst3, out, sbuf, dbuf, rows, zbuf, shared, gsem, ssem):
    c = lax.axis_index("c")
    s = lax.axis_index("s")

    def zloop(i, _):
        zbuf[i, :] = jnp.zeros((16,), jnp.float32)
        return 0
    lax.fori_loop(0, _ZCH, zloop, 0)
    for k in range(_NZC):
        pltpu.sync_copy(zbuf, shared.at[pl.ds(s * (_ZCH * _NZC) + k * _ZCH, _ZCH)])
    plsc.subcore_barrier()

    def macro(m, _):
        row0 = s * _ROWS_PT + m * _MROWS
        pltpu.sync_copy(src2.at[c, pl.ds(row0, _MROWS)], sbuf)
        pltpu.sync_copy(dst3.at[pl.ds(row0, _MROWS)], dbuf)
        gs = [pltpu.async_copy(h2.at[sbuf.at[j]], rows.at[j], gsem)
              for j in range(_MROWS)]
        for g in gs:
            g.wait()
        ss = [pltpu.async_copy(rows.at[j], shared.at[dbuf.at[j]], ssem,
                               add=True)
              for j in range(_MROWS)]
        for t in ss:
            t.wait()
        return 0
    lax.fori_loop(0, _NMACRO, macro, 0)
    plsc.subcore_barrier()

    npt = 6256

    @pl.when(s < _NSUB - 1)
    def _():
        pltpu.sync_copy(shared.at[pl.ds(s * npt, npt)],
                        out.at[c, pl.ds(s * npt, npt)])

    @pl.when(s == _NSUB - 1)
    def _():
        pltpu.sync_copy(shared.at[pl.ds((_NSUB - 1) * npt, N - (_NSUB - 1) * npt)],
                        out.at[c, pl.ds((_NSUB - 1) * npt, N - (_NSUB - 1) * npt)])


@jax.jit
def _sc_aggregate(h2, src2, dst3):
    mesh = plsc.VectorSubcoreMesh(core_axis_name="c", subcore_axis_name="s",
                                  num_cores=_NCORE, num_subcores=_NSUB)
    f = pl.kernel(
        _agg_body,
        out_type=jax.ShapeDtypeStruct((2, N, 16), jnp.float32),
        mesh=mesh,
        compiler_params=pltpu.CompilerParams(use_tc_tiling_on_sc=False),
        scratch_types=[
            pltpu.VMEM((_MROWS, _B), jnp.int32),
            pltpu.VMEM((_MROWS, _B), jnp.int32),
            pltpu.VMEM((_MROWS, _B, 16), jnp.float32),
            pltpu.VMEM((_ZCH, 16), jnp.float32),
            pltpu.VMEM_SHARED((_NSH, 16), jnp.float32),
            pltpu.SemaphoreType.DMA,
            pltpu.SemaphoreType.DMA,
        ],
    )
    return f(h2, src2, dst3)


def _head_body(xa_ref, xb_ref, fc1w_ref, fc1b_ref, fc2w_ref, fc2b_ref,
               outw_ref, outb_ref, o_ref):
    xc = xa_ref[...] + xb_ref[...]
    xc = jnp.maximum(jnp.dot(xc, fc1w_ref[...],
                             preferred_element_type=jnp.float32)
                     + fc1b_ref[...], 0.0)
    xc = jnp.maximum(jnp.dot(xc, fc2w_ref[...],
                             preferred_element_type=jnp.float32)
                     + fc2b_ref[...], 0.0)
    o_ref[...] = (jnp.dot(xc, outw_ref[...],
                          preferred_element_type=jnp.float32)
                  + outb_ref[...])


def _head(xa, xb, fc1_W, fc1_b, fc2_W, fc2_b, out_W, out_b):
    outw_p = jnp.zeros((256, 128), jnp.float32).at[:, :2].set(out_W)
    outb_p = jnp.zeros((128,), jnp.float32).at[:2].set(out_b)
    res = pl.pallas_call(
        _head_body,
        out_shape=jax.ShapeDtypeStruct((G, 128), jnp.float32),
    )(xa, xb, fc1_W, fc1_b.reshape(1, -1), fc2_W, fc2_b.reshape(1, -1),
      outw_p, outb_p.reshape(1, -1))
    return res[:, :2]


def _bn(x, g, b):
    m = jnp.mean(x, axis=0)
    v = jnp.var(x, axis=0)
    return (x - m) / jnp.sqrt(v + 1e-5) * g + b


def _gin_branch(x, src2, dst3, batch, p):
    h = x
    for i in range(5):
        hp = jnp.zeros((N, 32), jnp.float32).at[:, :31].set(h)
        h2 = jnp.concatenate([hp[:, :16], hp[:, 16:]], axis=0)
        agg3 = _sc_aggregate(h2, src2, dst3)
        aggp = jnp.concatenate([agg3[0], agg3[1]], axis=1)
        z = h + aggp[:, :31]
        z = jnp.maximum(z @ p[f'cW1_{i}'] + p[f'cb1_{i}'], 0.0)
        z = z @ p[f'cW2_{i}'] + p[f'cb2_{i}']
        h = jnp.maximum(z, 0.0)
        h = _bn(h, p[f'bng_{i}'], p[f'bnb_{i}'])
    pooled = jax.ops.segment_sum(h, batch, num_segments=G)
    return jnp.maximum(pooled @ p['fcxd_W'] + p['fcxd_b'], 0.0)


def _prep_edges(edge_index):
    src = edge_index[0].astype(jnp.int32)
    dst = edge_index[1].astype(jnp.int32)
    pad = _EPAD - E
    srcp = jnp.concatenate([src, jnp.zeros((pad,), jnp.int32)])
    dstp = jnp.concatenate([dst, jnp.full((pad,), N, jnp.int32)])
    src2 = jnp.stack([srcp, srcp + N]).reshape(2, _EROWS, _B)
    dst3 = dstp.reshape(_EROWS, _B)
    return src2, dst3


def kernel(x_a, edge_index_a, batch_a, x_b, edge_index_b, batch_b,
           cW1_0, cb1_0, cW2_0, cb2_0, bng_0, bnb_0,
           cW1_1, cb1_1, cW2_1, cb2_1, bng_1, bnb_1,
           cW1_2, cb1_2, cW2_2, cb2_2, bng_2, bnb_2,
           cW1_3, cb1_3, cW2_3, cb2_3, bng_3, bnb_3,
           cW1_4, cb1_4, cW2_4, cb2_4, bng_4, bnb_4,
           fcxd_W, fcxd_b, fc1_W, fc1_b, fc2_W, fc2_b,
           out_W, out_b):
    kw = dict(locals())
    src2_a, dst3_a = _prep_edges(edge_index_a)
    src2_b, dst3_b = _prep_edges(edge_index_b)
    xa = _gin_branch(x_a, src2_a, dst3_a, batch_a, kw)
    xb = _gin_branch(x_b, src2_b, dst3_b, batch_b, kw)
    return _head(xa, xb, fc1_W, fc1_b, fc2_W, fc2_b, out_W, out_b)
